# Initial kernel scaffold; baseline (speedup 1.0000x reference)
import jax
import jax.numpy as jnp
from jax.experimental import pallas as pl


def kernel(x_a, edge_index_a, batch_a, x_b, edge_index_b, batch_b, W1, b1, W2, b2, W3, b3):
    raise NotImplementedError("write your pallas kernel here")


# speedup vs baseline: 1.9346x; 1.9346x over previous
import jax
import jax.numpy as jnp
from jax import lax
from jax.experimental import pallas as pl
from jax.experimental.pallas import tpu as pltpu, tpu_sc as plsc

N = 10000
E = 320000
D = 128
G = 8
NC = 2
NS = 16
NW = NC * NS
CHUNK = 128
BATCH = 2
NP = 10240
NCH = 80
EPAD = NW * NCH * CHUNK
NROW = EPAD // CHUNK
ROUNDS = NCH // BATCH
PAD_ROW = 10200
SLAB = NP // NS

_MESH = dict(core_axis_name="c", subcore_axis_name="s")


def _agg_body(gflat_hbm, src_hbm, dst_hbm, zeros_hbm, out_hbm,
              gi_v, di_v, rows_v, sem, acc):
    cid = lax.axis_index("c")
    sid = lax.axis_index("s")
    wid = cid * NS + sid
    pltpu.sync_copy(zeros_hbm, rows_v.at[pl.ds(0, CHUNK)])
    for k in range(SLAB // CHUNK):
        r0 = pl.multiple_of(sid * SLAB + k * CHUNK, CHUNK)
        pltpu.sync_copy(rows_v.at[pl.ds(0, CHUNK)], acc.at[pl.ds(r0, CHUNK)])
    plsc.subcore_barrier()

    def rnd(r, _):
        row0 = pl.multiple_of(wid * NCH + r * BATCH, BATCH)
        pltpu.sync_copy(src_hbm.at[pl.ds(row0, BATCH)], gi_v)
        pltpu.sync_copy(dst_hbm.at[pl.ds(row0, BATCH)], di_v)
        cps = [pltpu.async_copy(gflat_hbm.at[gi_v.at[k]],
                                rows_v.at[pl.ds(k * CHUNK, CHUNK)], sem)
               for k in range(BATCH)]
        for cp in cps:
            cp.wait()
        for t in range(NS):
            @pl.when(sid == t)
            def _():
                for k in range(BATCH):
                    pltpu.sync_copy(rows_v.at[pl.ds(k * CHUNK, CHUNK)],
                                    acc.at[di_v.at[k]], add=True)
            plsc.subcore_barrier()
        return 0

    lax.fori_loop(0, ROUNDS, rnd, 0)
    r0 = pl.multiple_of(sid * SLAB, SLAB)
    pltpu.sync_copy(acc.at[pl.ds(r0, SLAB)],
                    out_hbm.at[cid, pl.ds(r0, SLAB)])


def _sc_aggregate(gflat, src, dst, zc):
    return pl.kernel(
        _agg_body,
        out_type=jax.ShapeDtypeStruct((NC, NP, D), jnp.float32),
        mesh=plsc.VectorSubcoreMesh(**_MESH),
        scratch_types=[
            pltpu.VMEM((BATCH, CHUNK), jnp.int32),
            pltpu.VMEM((BATCH, CHUNK), jnp.int32),
            pltpu.VMEM((BATCH * CHUNK, D), jnp.float32),
            pltpu.SemaphoreType.DMA,
            pltpu.VMEM_SHARED((NP, D), jnp.float32),
        ],
    )(gflat, src, dst, zc)


def _deg_body(dst_hbm, zeros_hbm, ones_hbm, out_hbm, di_v, rows_v, acc):
    cid = lax.axis_index("c")
    sid = lax.axis_index("s")
    wid = cid * NS + sid
    pltpu.sync_copy(zeros_hbm, rows_v.at[pl.ds(0, CHUNK)])
    for k in range(SLAB // CHUNK):
        r0 = pl.multiple_of(sid * SLAB + k * CHUNK, CHUNK)
        pltpu.sync_copy(rows_v.at[pl.ds(0, CHUNK)], acc.at[pl.ds(r0, CHUNK)])
    for k in range(BATCH):
        pltpu.sync_copy(ones_hbm, rows_v.at[pl.ds(k * CHUNK, CHUNK)])
    plsc.subcore_barrier()

    def rnd(r, _):
        row0 = pl.multiple_of(wid * NCH + r * BATCH, BATCH)
        pltpu.sync_copy(dst_hbm.at[pl.ds(row0, BATCH)], di_v)
        for t in range(NS):
            @pl.when(sid == t)
            def _():
                for k in range(BATCH):
                    pltpu.sync_copy(rows_v.at[pl.ds(k * CHUNK, CHUNK)],
                                    acc.at[di_v.at[k]], add=True)
            plsc.subcore_barrier()
        return 0

    lax.fori_loop(0, ROUNDS, rnd, 0)
    r0 = pl.multiple_of(sid * SLAB, SLAB)
    pltpu.sync_copy(acc.at[pl.ds(r0, SLAB)],
                    out_hbm.at[cid, pl.ds(r0, SLAB)])


def _sc_degrees(dst, zc, oc):
    return pl.kernel(
        _deg_body,
        out_type=jax.ShapeDtypeStruct((NC, NP, D), jnp.float32),
        mesh=plsc.VectorSubcoreMesh(**_MESH),
        scratch_types=[
            pltpu.VMEM((BATCH, CHUNK), jnp.int32),
            pltpu.VMEM((BATCH * CHUNK, D), jnp.float32),
            pltpu.VMEM_SHARED((NP, D), jnp.float32),
        ],
    )(dst, zc, oc)


_RB = 512
_NB = NP // _RB


def _prep_body(deg_ref, x_ref, w_ref, g_ref, dis_ref):
    deg = (deg_ref[0, :, 0:1] + deg_ref[1, :, 0:1]) + 1.0
    dis = lax.rsqrt(deg)
    dis_ref[...] = dis
    g_ref[...] = dis * jnp.dot(x_ref[...], w_ref[...],
                               preferred_element_type=jnp.float32)


def _tc_prep(deg, x, w1):
    return pl.pallas_call(
        _prep_body,
        grid=(_NB,),
        in_specs=[
            pl.BlockSpec((NC, _RB, D), lambda i: (0, i, 0)),
            pl.BlockSpec((_RB, D), lambda i: (i, 0)),
            pl.BlockSpec((D, D), lambda i: (0, 0)),
        ],
        out_specs=[
            pl.BlockSpec((_RB, D), lambda i: (i, 0)),
            pl.BlockSpec((_RB, 1), lambda i: (i, 0)),
        ],
        out_shape=[
            jax.ShapeDtypeStruct((NP, D), jnp.float32),
            jax.ShapeDtypeStruct((NP, 1), jnp.float32),
        ],
    )(deg, x, w1)


def _mid_body(agg_ref, g_ref, dis_ref, b_ref, w_ref, out_ref):
    dis = dis_ref[...]
    h = dis * (agg_ref[0] + agg_ref[1] + g_ref[...]) + b_ref[...]
    h = jnp.maximum(h, 0.0)
    out_ref[...] = dis * jnp.dot(h, w_ref[...],
                                 preferred_element_type=jnp.float32)


def _tc_mid(agg, g, dis, b, w):
    return pl.pallas_call(
        _mid_body,
        grid=(_NB,),
        in_specs=[
            pl.BlockSpec((NC, _RB, D), lambda i: (0, i, 0)),
            pl.BlockSpec((_RB, D), lambda i: (i, 0)),
            pl.BlockSpec((_RB, 1), lambda i: (i, 0)),
            pl.BlockSpec((1, D), lambda i: (0, 0)),
            pl.BlockSpec((D, D), lambda i: (0, 0)),
        ],
        out_specs=pl.BlockSpec((_RB, D), lambda i: (i, 0)),
        out_shape=jax.ShapeDtypeStruct((NP, D), jnp.float32),
    )(agg, g, dis, b, w)


def _fin_body(agg_a_ref, g_a_ref, dis_a_ref, bat_a_ref,
              agg_b_ref, g_b_ref, dis_b_ref, bat_b_ref, b3_ref,
              out_ref, sa, ca, sb, cb):
    i = pl.program_id(0)
    ones = jnp.ones((_RB, D), jnp.float32)
    gids = lax.broadcasted_iota(jnp.int32, (1, G), 1)
    dn = (((0,), (0,)), ((), ()))

    def branch(agg_ref, g_ref, dis_ref, bat_ref):
        out3 = (dis_ref[...] * (agg_ref[0] + agg_ref[1] + g_ref[...])
                + b3_ref[...])
        p = (bat_ref[...] == gids).astype(jnp.float32)
        s = lax.dot_general(p, out3, dn,
                            preferred_element_type=jnp.float32)
        c = lax.dot_general(p, ones, dn,
                            preferred_element_type=jnp.float32)
        return s, c

    s_a, c_a = branch(agg_a_ref, g_a_ref, dis_a_ref, bat_a_ref)
    s_b, c_b = branch(agg_b_ref, g_b_ref, dis_b_ref, bat_b_ref)

    @pl.when(i == 0)
    def _():
        sa[...] = s_a
        ca[...] = c_a
        sb[...] = s_b
        cb[...] = c_b

    @pl.when(i > 0)
    def _():
        sa[...] += s_a
        ca[...] += c_a
        sb[...] += s_b
        cb[...] += c_b

    @pl.when(i == _NB - 1)
    def _():
        mean_a = sa[...] / jnp.maximum(ca[...], 1.0)
        mean_b = sb[...] / jnp.maximum(cb[...], 1.0)
        diff = mean_a - mean_b + 1e-6
        out_ref[...] = jnp.sqrt(jnp.sum(diff * diff, axis=1, keepdims=True))


def _tc_final(agg_a, g_a, dis_a, bat_a, agg_b, g_b, dis_b, bat_b, b3):
    node_specs = lambda: [
        pl.BlockSpec((NC, _RB, D), lambda i: (0, i, 0)),
        pl.BlockSpec((_RB, D), lambda i: (i, 0)),
        pl.BlockSpec((_RB, 1), lambda i: (i, 0)),
        pl.BlockSpec((_RB, 1), lambda i: (i, 0)),
    ]
    return pl.pallas_call(
        _fin_body,
        grid=(_NB,),
        in_specs=node_specs() + node_specs() + [
            pl.BlockSpec((1, D), lambda i: (0, 0)),
        ],
        out_specs=pl.BlockSpec((G, 1), lambda i: (0, 0)),
        out_shape=jax.ShapeDtypeStruct((G, 1), jnp.float32),
        scratch_shapes=[pltpu.VMEM((G, D), jnp.float32)] * 4,
    )(agg_a, g_a, dis_a, bat_a, agg_b, g_b, dis_b, bat_b, b3)


def _pad_edges(edge_index):
    pad = EPAD - E
    src = jnp.concatenate([edge_index[0].astype(jnp.int32),
                             jnp.zeros((pad,), jnp.int32)])
    dst = jnp.concatenate([edge_index[1].astype(jnp.int32),
                           jnp.full((pad,), PAD_ROW, jnp.int32)])
    return src.reshape(NROW, CHUNK), dst.reshape(NROW, CHUNK)


def kernel(x_a, edge_index_a, batch_a, x_b, edge_index_b, batch_b,
           W1, b1, W2, b2, W3, b3):
    src_a, dst_a = _pad_edges(edge_index_a)
    src_b, dst_b = _pad_edges(edge_index_b)
    padg = jnp.full((NP - N, 1), G, jnp.int32)
    bat_a = jnp.concatenate([batch_a.astype(jnp.int32).reshape(N, 1), padg])
    bat_b = jnp.concatenate([batch_b.astype(jnp.int32).reshape(N, 1), padg])
    padx = jnp.zeros((NP - N, D), jnp.float32)
    x_a = jnp.concatenate([x_a, padx])
    x_b = jnp.concatenate([x_b, padx])
    b1r = b1.reshape(1, D)
    b2r = b2.reshape(1, D)
    b3r = b3.reshape(1, D)
    zc = jnp.zeros((CHUNK, D), jnp.float32)
    oc = jnp.ones((CHUNK, D), jnp.float32)

    deg_a = _sc_degrees(dst_a, zc, oc)
    deg_b = _sc_degrees(dst_b, zc, oc)

    def encode(x, src, dst, deg):
        g1, dis = _tc_prep(deg, x, W1)
        agg1 = _sc_aggregate(g1, src, dst, zc)
        g2 = _tc_mid(agg1, g1, dis, b1r, W2)
        agg2 = _sc_aggregate(g2, src, dst, zc)
        g3 = _tc_mid(agg2, g2, dis, b2r, W3)
        agg3 = _sc_aggregate(g3, src, dst, zc)
        return agg3, g3, dis

    agg_a, g_a, dis_a = encode(x_a, src_a, dst_a, deg_a)
    agg_b, g_b, dis_b = encode(x_b, src_b, dst_b, deg_b)

    out = _tc_final(agg_a, g_a, dis_a, bat_a, agg_b, g_b, dis_b, bat_b, b3r)
    return out.reshape(G)


# speedup vs baseline: 2.4134x; 1.2475x over previous
import jax
import jax.numpy as jnp
from jax import lax
from jax.experimental import pallas as pl
from jax.experimental.pallas import tpu as pltpu, tpu_sc as plsc

N = 10000
E = 320000
D = 128
G = 8
NC = 2
NS = 16
NW = NC * NS
CHUNK = 128
BATCH = 2
NP = 10240
NCH = 80
EPAD = NW * NCH * CHUNK
NROW = EPAD // CHUNK
ROUNDS = NCH // BATCH
PAD_ROW = 10200
SLAB = NP // NS

_MESH = dict(core_axis_name="c", subcore_axis_name="s")


def _agg_body(gflat_hbm, src_hbm, dst_hbm, zeros_hbm, out_hbm,
              gi_v, di_v, rows_v, sem, acc):
    cid = lax.axis_index("c")
    sid = lax.axis_index("s")
    wid = cid * NS + sid
    pltpu.sync_copy(zeros_hbm, rows_v.at[0])
    for k in range(SLAB // CHUNK):
        r0 = pl.multiple_of(sid * SLAB + k * CHUNK, CHUNK)
        pltpu.sync_copy(rows_v.at[0], acc.at[pl.ds(r0, CHUNK)])
    plsc.subcore_barrier()

    row_base = pl.multiple_of(wid * NCH, NCH)
    pltpu.sync_copy(src_hbm.at[row_base], gi_v.at[0])
    pltpu.sync_copy(dst_hbm.at[row_base], di_v.at[0])
    pltpu.async_copy(gflat_hbm.at[gi_v.at[0]], rows_v.at[0], sem)

    def rnd(r, _):
        cur = lax.rem(r, 2)
        nxt = lax.rem(r + 1, 2)
        pltpu.make_async_copy(gflat_hbm.at[gi_v.at[cur]],
                              rows_v.at[cur], sem).wait()

        @pl.when(r + 1 < NCH)
        def _():
            pltpu.sync_copy(src_hbm.at[row_base + r + 1], gi_v.at[nxt])
            pltpu.sync_copy(dst_hbm.at[row_base + r + 1], di_v.at[nxt])
            pltpu.async_copy(gflat_hbm.at[gi_v.at[nxt]], rows_v.at[nxt], sem)

        for t in range(NS):
            @pl.when(sid == t)
            def _():
                pltpu.sync_copy(rows_v.at[cur], acc.at[di_v.at[cur]],
                                add=True)
            plsc.subcore_barrier()
        return 0

    lax.fori_loop(0, NCH, rnd, 0)
    r0 = pl.multiple_of(sid * SLAB, SLAB)
    pltpu.sync_copy(acc.at[pl.ds(r0, SLAB)],
                    out_hbm.at[cid, pl.ds(r0, SLAB)])


def _sc_aggregate(gflat, src, dst, zc):
    return pl.kernel(
        _agg_body,
        out_type=jax.ShapeDtypeStruct((NC, NP, D), jnp.float32),
        mesh=plsc.VectorSubcoreMesh(**_MESH),
        scratch_types=[
            pltpu.VMEM((2, CHUNK), jnp.int32),
            pltpu.VMEM((2, CHUNK), jnp.int32),
            pltpu.VMEM((2, CHUNK, D), jnp.float32),
            pltpu.SemaphoreType.DMA,
            pltpu.VMEM_SHARED((NP, D), jnp.float32),
        ],
    )(gflat, src, dst, zc)


def _deg_body(dst_hbm, zeros_hbm, ones_hbm, out_hbm, di_v, rows_v, acc):
    cid = lax.axis_index("c")
    sid = lax.axis_index("s")
    wid = cid * NS + sid
    pltpu.sync_copy(zeros_hbm, rows_v.at[pl.ds(0, CHUNK)])
    for k in range(SLAB // CHUNK):
        r0 = pl.multiple_of(sid * SLAB + k * CHUNK, CHUNK)
        pltpu.sync_copy(rows_v.at[pl.ds(0, CHUNK)], acc.at[pl.ds(r0, CHUNK)])
    for k in range(BATCH):
        pltpu.sync_copy(ones_hbm, rows_v.at[pl.ds(k * CHUNK, CHUNK)])
    plsc.subcore_barrier()

    def rnd(r, _):
        row0 = pl.multiple_of(wid * NCH + r * BATCH, BATCH)
        pltpu.sync_copy(dst_hbm.at[pl.ds(row0, BATCH)], di_v)
        for t in range(NS):
            @pl.when(sid == t)
            def _():
                for k in range(BATCH):
                    pltpu.sync_copy(rows_v.at[pl.ds(k * CHUNK, CHUNK)],
                                    acc.at[di_v.at[k]], add=True)
            plsc.subcore_barrier()
        return 0

    lax.fori_loop(0, ROUNDS, rnd, 0)
    r0 = pl.multiple_of(sid * SLAB, SLAB)
    pltpu.sync_copy(acc.at[pl.ds(r0, SLAB)],
                    out_hbm.at[cid, pl.ds(r0, SLAB)])


def _sc_degrees(dst, zc, oc):
    return pl.kernel(
        _deg_body,
        out_type=jax.ShapeDtypeStruct((NC, NP, D), jnp.float32),
        mesh=plsc.VectorSubcoreMesh(**_MESH),
        scratch_types=[
            pltpu.VMEM((BATCH, CHUNK), jnp.int32),
            pltpu.VMEM((BATCH * CHUNK, D), jnp.float32),
            pltpu.VMEM_SHARED((NP, D), jnp.float32),
        ],
    )(dst, zc, oc)


_RB = 512
_NB = NP // _RB


def _prep_body(deg_ref, x_ref, w_ref, g_ref, dis_ref):
    deg = (deg_ref[0, :, 0:1] + deg_ref[1, :, 0:1]) + 1.0
    dis = lax.rsqrt(deg)
    dis_ref[...] = dis
    g_ref[...] = dis * jnp.dot(x_ref[...], w_ref[...],
                               preferred_element_type=jnp.float32)


def _tc_prep(deg, x, w1):
    return pl.pallas_call(
        _prep_body,
        grid=(_NB,),
        in_specs=[
            pl.BlockSpec((NC, _RB, D), lambda i: (0, i, 0)),
            pl.BlockSpec((_RB, D), lambda i: (i, 0)),
            pl.BlockSpec((D, D), lambda i: (0, 0)),
        ],
        out_specs=[
            pl.BlockSpec((_RB, D), lambda i: (i, 0)),
            pl.BlockSpec((_RB, 1), lambda i: (i, 0)),
        ],
        out_shape=[
            jax.ShapeDtypeStruct((NP, D), jnp.float32),
            jax.ShapeDtypeStruct((NP, 1), jnp.float32),
        ],
    )(deg, x, w1)


def _mid_body(agg_ref, g_ref, dis_ref, b_ref, w_ref, out_ref):
    dis = dis_ref[...]
    h = dis * (agg_ref[0] + agg_ref[1] + g_ref[...]) + b_ref[...]
    h = jnp.maximum(h, 0.0)
    out_ref[...] = dis * jnp.dot(h, w_ref[...],
                                 preferred_element_type=jnp.float32)


def _tc_mid(agg, g, dis, b, w):
    return pl.pallas_call(
        _mid_body,
        grid=(_NB,),
        in_specs=[
            pl.BlockSpec((NC, _RB, D), lambda i: (0, i, 0)),
            pl.BlockSpec((_RB, D), lambda i: (i, 0)),
            pl.BlockSpec((_RB, 1), lambda i: (i, 0)),
            pl.BlockSpec((1, D), lambda i: (0, 0)),
            pl.BlockSpec((D, D), lambda i: (0, 0)),
        ],
        out_specs=pl.BlockSpec((_RB, D), lambda i: (i, 0)),
        out_shape=jax.ShapeDtypeStruct((NP, D), jnp.float32),
    )(agg, g, dis, b, w)


def _fin_body(agg_a_ref, g_a_ref, dis_a_ref, bat_a_ref,
              agg_b_ref, g_b_ref, dis_b_ref, bat_b_ref, b3_ref,
              out_ref, sa, ca, sb, cb):
    i = pl.program_id(0)
    ones = jnp.ones((_RB, D), jnp.float32)
    gids = lax.broadcasted_iota(jnp.int32, (1, G), 1)
    dn = (((0,), (0,)), ((), ()))

    def branch(agg_ref, g_ref, dis_ref, bat_ref):
        out3 = (dis_ref[...] * (agg_ref[0] + agg_ref[1] + g_ref[...])
                + b3_ref[...])
        p = (bat_ref[...] == gids).astype(jnp.float32)
        s = lax.dot_general(p, out3, dn,
                            preferred_element_type=jnp.float32)
        c = lax.dot_general(p, ones, dn,
                            preferred_element_type=jnp.float32)
        return s, c

    s_a, c_a = branch(agg_a_ref, g_a_ref, dis_a_ref, bat_a_ref)
    s_b, c_b = branch(agg_b_ref, g_b_ref, dis_b_ref, bat_b_ref)

    @pl.when(i == 0)
    def _():
        sa[...] = s_a
        ca[...] = c_a
        sb[...] = s_b
        cb[...] = c_b

    @pl.when(i > 0)
    def _():
        sa[...] += s_a
        ca[...] += c_a
        sb[...] += s_b
        cb[...] += c_b

    @pl.when(i == _NB - 1)
    def _():
        mean_a = sa[...] / jnp.maximum(ca[...], 1.0)
        mean_b = sb[...] / jnp.maximum(cb[...], 1.0)
        diff = mean_a - mean_b + 1e-6
        out_ref[...] = jnp.sqrt(jnp.sum(diff * diff, axis=1, keepdims=True))


def _tc_final(agg_a, g_a, dis_a, bat_a, agg_b, g_b, dis_b, bat_b, b3):
    node_specs = lambda: [
        pl.BlockSpec((NC, _RB, D), lambda i: (0, i, 0)),
        pl.BlockSpec((_RB, D), lambda i: (i, 0)),
        pl.BlockSpec((_RB, 1), lambda i: (i, 0)),
        pl.BlockSpec((_RB, 1), lambda i: (i, 0)),
    ]
    return pl.pallas_call(
        _fin_body,
        grid=(_NB,),
        in_specs=node_specs() + node_specs() + [
            pl.BlockSpec((1, D), lambda i: (0, 0)),
        ],
        out_specs=pl.BlockSpec((G, 1), lambda i: (0, 0)),
        out_shape=jax.ShapeDtypeStruct((G, 1), jnp.float32),
        scratch_shapes=[pltpu.VMEM((G, D), jnp.float32)] * 4,
    )(agg_a, g_a, dis_a, bat_a, agg_b, g_b, dis_b, bat_b, b3)


def _pad_edges(edge_index):
    pad = EPAD - E
    src = jnp.concatenate([edge_index[0].astype(jnp.int32),
                             jnp.zeros((pad,), jnp.int32)])
    dst = jnp.concatenate([edge_index[1].astype(jnp.int32),
                           jnp.full((pad,), PAD_ROW, jnp.int32)])
    return src.reshape(NROW, CHUNK), dst.reshape(NROW, CHUNK)


def kernel(x_a, edge_index_a, batch_a, x_b, edge_index_b, batch_b,
           W1, b1, W2, b2, W3, b3):
    src_a, dst_a = _pad_edges(edge_index_a)
    src_b, dst_b = _pad_edges(edge_index_b)
    padg = jnp.full((NP - N, 1), G, jnp.int32)
    bat_a = jnp.concatenate([batch_a.astype(jnp.int32).reshape(N, 1), padg])
    bat_b = jnp.concatenate([batch_b.astype(jnp.int32).reshape(N, 1), padg])
    padx = jnp.zeros((NP - N, D), jnp.float32)
    x_a = jnp.concatenate([x_a, padx])
    x_b = jnp.concatenate([x_b, padx])
    b1r = b1.reshape(1, D)
    b2r = b2.reshape(1, D)
    b3r = b3.reshape(1, D)
    zc = jnp.zeros((CHUNK, D), jnp.float32)
    oc = jnp.ones((CHUNK, D), jnp.float32)

    deg_a = _sc_degrees(dst_a, zc, oc)
    deg_b = _sc_degrees(dst_b, zc, oc)

    def encode(x, src, dst, deg):
        g1, dis = _tc_prep(deg, x, W1)
        agg1 = _sc_aggregate(g1, src, dst, zc)
        g2 = _tc_mid(agg1, g1, dis, b1r, W2)
        agg2 = _sc_aggregate(g2, src, dst, zc)
        g3 = _tc_mid(agg2, g2, dis, b2r, W3)
        agg3 = _sc_aggregate(g3, src, dst, zc)
        return agg3, g3, dis

    agg_a, g_a, dis_a = encode(x_a, src_a, dst_a, deg_a)
    agg_b, g_b, dis_b = encode(x_b, src_b, dst_b, deg_b)

    out = _tc_final(agg_a, g_a, dis_a, bat_a, agg_b, g_b, dis_b, bat_b, b3r)
    return out.reshape(G)
